# Initial kernel scaffold; baseline (speedup 1.0000x reference)
import jax
import jax.numpy as jnp
from jax.experimental import pallas as pl


def kernel(x, edge_index, e_id, edge_weight, W_rel1, b_rel1, W_root1, W_rel2, b_rel2, W_root2):
    raise NotImplementedError("write your pallas kernel here")


# speedup vs baseline: 3.7704x; 3.7704x over previous
import functools

import jax
import jax.numpy as jnp
from jax import lax
from jax.experimental import pallas as pl
from jax.experimental.pallas import tpu as pltpu
from jax.experimental.pallas import tpu_sc as plsc

_NC = 2
_NS = 16
_NW = _NC * _NS
_CHUNK = 128


@functools.lru_cache(maxsize=None)
def _build_sc_agg(N, D, E):
    assert E % _CHUNK == 0
    n_chunks = E // _CHUNK
    nj = (n_chunks + _NW - 1) // _NW
    rows_per_tile = ((N + _NS - 1) // _NS + 7) // 8 * 8
    N_pad = rows_per_tile * _NS
    zstep = rows_per_tile
    while zstep > _CHUNK or rows_per_tile % zstep:
        zstep //= 2
    if rows_per_tile % zstep:
        zstep = 8
    assert rows_per_tile % zstep == 0 and zstep <= _CHUNK

    mesh = plsc.VectorSubcoreMesh(core_axis_name="c", subcore_axis_name="s")

    @functools.partial(
        pl.kernel,
        mesh=mesh,
        out_type=jax.ShapeDtypeStruct((_NC * N_pad, D), jnp.float32),
        scratch_types=[
            pltpu.VMEM((_CHUNK,), jnp.int32),
            pltpu.VMEM((_CHUNK,), jnp.int32),
            pltpu.VMEM((_CHUNK,), jnp.int32),
            pltpu.VMEM((_CHUNK,), jnp.float32),
            pltpu.VMEM((_CHUNK, D), jnp.float32),
            pltpu.VMEM_SHARED((N_pad, D), jnp.float32),
            pltpu.SemaphoreType.DMA,
        ],
    )
    def sc_agg(x_hbm, src_hbm, dst_hbm, eid_hbm, ew_hbm, out_hbm,
               src_v, dst_v, eid_v, ew_v, rows_v, acc, sem):
        c = lax.axis_index("c")
        s = lax.axis_index("s")
        wid = s * _NC + c

        zeros16 = jnp.zeros((16,), jnp.float32)

        def zrow(i, _):
            for jj in range(D // 16):
                rows_v[i, pl.ds(jj * 16, 16)] = zeros16
            return 0

        lax.fori_loop(0, zstep, zrow, 0)
        base_row = s * rows_per_tile

        def zcopy(i, _):
            pltpu.sync_copy(rows_v.at[pl.ds(0, zstep)],
                            acc.at[pl.ds(base_row + i * zstep, zstep)])
            return 0

        lax.fori_loop(0, rows_per_tile // zstep, zcopy, 0)
        plsc.subcore_barrier()

        def chunk_body(j, _):
            chunk = j * _NW + wid

            @pl.when(chunk < n_chunks)
            def _():
                base = chunk * _CHUNK
                pltpu.sync_copy(src_hbm.at[pl.ds(base, _CHUNK)], src_v)
                pltpu.sync_copy(dst_hbm.at[pl.ds(base, _CHUNK)], dst_v)
                pltpu.sync_copy(eid_hbm.at[pl.ds(base, _CHUNK)], eid_v)
                pltpu.async_copy(ew_hbm.at[eid_v], ew_v, sem).wait()
                pltpu.async_copy(x_hbm.at[src_v], rows_v, sem).wait()

                def scale16(g, _):
                    w16 = ew_v[pl.ds(g * 16, 16)]
                    for kk in range(16):
                        k = g * 16 + kk
                        w = jnp.take(w16, jnp.full((16,), kk, jnp.int32))
                        for jj in range(D // 16):
                            sl = pl.ds(jj * 16, 16)
                            rows_v[k, sl] = rows_v[k, sl] * w
                    return 0

                lax.fori_loop(0, _CHUNK // 16, scale16, 0)
                pltpu.sync_copy(rows_v, acc.at[dst_v], add=True)

            return 0

        lax.fori_loop(0, nj, chunk_body, 0)
        plsc.subcore_barrier()

        pltpu.sync_copy(acc.at[pl.ds(base_row, rows_per_tile)],
                        out_hbm.at[pl.ds(c * N_pad + base_row, rows_per_tile)])

    return sc_agg


@functools.partial(jax.jit, static_argnames=("act",))
def _tc_combine(p0, p1, x, WrT, br, WroT, act):
    N, D = x.shape
    BR = 1000
    nb = N // BR
    assert nb * BR == N

    def body(p0_ref, p1_ref, x_ref, wr_ref, br_ref, wro_ref, o_ref):
        agg = p0_ref[...] + p1_ref[...]
        h = jnp.dot(agg, wr_ref[...], preferred_element_type=jnp.float32)
        h = h + jnp.dot(x_ref[...], wro_ref[...],
                        preferred_element_type=jnp.float32)
        h = h + br_ref[...]
        o_ref[...] = jnp.tanh(h) if act else h

    return pl.pallas_call(
        body,
        grid=(nb,),
        in_specs=[
            pl.BlockSpec((BR, D), lambda i: (i, 0)),
            pl.BlockSpec((BR, D), lambda i: (i, 0)),
            pl.BlockSpec((BR, D), lambda i: (i, 0)),
            pl.BlockSpec((D, D), lambda i: (0, 0)),
            pl.BlockSpec((1, D), lambda i: (0, 0)),
            pl.BlockSpec((D, D), lambda i: (0, 0)),
        ],
        out_specs=pl.BlockSpec((BR, D), lambda i: (i, 0)),
        out_shape=jax.ShapeDtypeStruct((N, D), jnp.float32),
    )(p0, p1, x, WrT, br, WroT)


def kernel(x, edge_index, e_id, edge_weight,
           W_rel1, b_rel1, W_root1, W_rel2, b_rel2, W_root2):
    N, D = x.shape
    E = e_id.shape[0]
    src = edge_index[0]
    dst = edge_index[1]
    sc_agg = _build_sc_agg(N, D, E)
    N_pad = ((N + _NS - 1) // _NS + 7) // 8 * 8 * _NS
    p = sc_agg(x, src, dst, e_id, edge_weight)
    h = _tc_combine(p[:N], p[N_pad:N_pad + N], x,
                    W_rel1.T, b_rel1[None, :], W_root1.T, act=False)
    p = sc_agg(h, src, dst, e_id, edge_weight)
    return _tc_combine(p[:N], p[N_pad:N_pad + N], h,
                       W_rel2.T, b_rel2[None, :], W_root2.T, act=True)
